# Initial kernel scaffold; baseline (speedup 1.0000x reference)
import jax
import jax.numpy as jnp
from jax.experimental import pallas as pl


def kernel(features, edge_index):
    raise NotImplementedError("write your pallas kernel here")


# speedup vs baseline: 10.2669x; 10.2669x over previous
import functools

import jax
import jax.numpy as jnp
from jax import lax
from jax.experimental import pallas as pl
from jax.experimental.pallas import tpu as pltpu
from jax.experimental.pallas import tpu_sc as plsc

N = 10000
E = 320000
D = 128
NC, NS = 2, 16
NW = NC * NS
RPT = 632
NPAD = NS * RPT
NDEG = 10016
EPW = E // NW
K = 80
NCHUNK = EPW // K


def _normalize_body(f_ref, out_ref):
    f = f_ref[...]
    norm = jnp.sqrt(jnp.sum(f * f, axis=1, keepdims=True))
    out_ref[...] = f / jnp.maximum(norm, 1e-12)


def _spmm_body(fnorm_hbm, src_hbm, dst_hbm, zeros_hbm, out_hbm, deg_hbm,
               src_v, dst_v, rows_v, deg_v, acc, sem):
    cid = lax.axis_index("c")
    sid = lax.axis_index("s")
    wid = sid * NC + cid

    pltpu.sync_copy(zeros_hbm.at[pl.ds(sid * RPT, RPT)],
                    acc.at[pl.ds(sid * RPT, RPT)])

    zero16 = jnp.zeros((16,), jnp.float32)

    def zbody(i, _):
        deg_v[pl.ds(i * 16, 16)] = zero16
        return ()

    lax.fori_loop(0, NDEG // 16, zbody, ())
    plsc.subcore_barrier()

    base = wid * EPW
    ones16 = jnp.ones((16,), jnp.float32)

    def body(i, _):
        off = base + i * K
        pltpu.sync_copy(src_hbm.at[pl.ds(off, K)], src_v)
        pltpu.sync_copy(dst_hbm.at[pl.ds(off, K)], dst_v)
        gather = pltpu.async_copy(fnorm_hbm.at[dst_v], rows_v, sem)
        for j in range(K // 16):
            idx = src_v[pl.ds(j * 16, 16)]
            plsc.addupdate_scatter(deg_v, [idx], ones16)
        gather.wait()
        pltpu.sync_copy(rows_v, acc.at[src_v], add=True)
        return ()

    lax.fori_loop(0, NCHUNK, body, ())

    pltpu.sync_copy(deg_v, deg_hbm.at[wid])
    plsc.subcore_barrier()
    pltpu.sync_copy(acc.at[pl.ds(sid * RPT, RPT)],
                    out_hbm.at[cid].at[pl.ds(sid * RPT, RPT)])


def _finish_body(part_ref, deg_ref, fnorm_ref, out_ref):
    acc = part_ref[0] + part_ref[1]
    s = acc[:N, :]
    fn = fnorm_ref[...]
    sim = jnp.sum(fn * s, axis=1, keepdims=True)
    deg = lax.dot_general(deg_ref[...], jnp.ones((NW, 1), jnp.float32),
                          (((0,), (0,)), ((), ())),
                          preferred_element_type=jnp.float32,
                          precision=lax.Precision.HIGHEST)
    deg = deg[:N, :]
    deg = jnp.where(deg == 0.0, 1.0, deg)
    score = (sim / deg + 1.0) * 0.5
    out_ref[...] = jnp.clip(score, 0.0, 1.0)


def kernel(features, edge_index):
    fnorm = pl.pallas_call(
        _normalize_body,
        out_shape=jax.ShapeDtypeStruct((N, D), jnp.float32),
    )(features)

    src = edge_index[0]
    dst = edge_index[1]
    zeros = jnp.zeros((NPAD, D), jnp.float32)

    mesh = plsc.VectorSubcoreMesh(core_axis_name="c", subcore_axis_name="s")
    spmm = functools.partial(
        pl.kernel,
        out_type=(
            jax.ShapeDtypeStruct((NC, NPAD, D), jnp.float32),
            jax.ShapeDtypeStruct((NW, NDEG), jnp.float32),
        ),
        mesh=mesh,
        compiler_params=pltpu.CompilerParams(needs_layout_passes=False),
        scratch_types=[
            pltpu.VMEM((K,), jnp.int32),
            pltpu.VMEM((K,), jnp.int32),
            pltpu.VMEM((K, D), jnp.float32),
            pltpu.VMEM((NDEG,), jnp.float32),
            pltpu.VMEM_SHARED((NPAD, D), jnp.float32),
            pltpu.SemaphoreType.DMA,
        ],
    )(_spmm_body)
    partials, deg_part = spmm(fnorm, src, dst, zeros)

    scores = pl.pallas_call(
        _finish_body,
        out_shape=jax.ShapeDtypeStruct((N, 1), jnp.float32),
    )(partials, deg_part, fnorm)
    return scores[:, 0]
